# Initial kernel scaffold; baseline (speedup 1.0000x reference)
import jax
import jax.numpy as jnp
from jax.experimental import pallas as pl


def kernel(x, edge_index, batch, root_ctx_norm, W0, a_s0, a_d0, b0, W1, a_s1, a_d1, b1, W2, a_s2, a_d2, b2, fc1_w, fc1_b, fc2_w, fc2_b, rh1_w, rh1_b, rh2_w, rh2_b, rh3_w, rh3_b):
    raise NotImplementedError("write your pallas kernel here")


# speedup vs baseline: 1.0443x; 1.0443x over previous
import functools

import jax
import jax.numpy as jnp
from jax.experimental import pallas as pl
from jax.experimental.pallas import tpu as pltpu

N = 98304
E = 188416
B = 4096
HEADS = 4
HID = 64
NF = 24
NJ = 24
TL = 15
CL = 11
GF = 3
ROT_OUT = NJ * TL * NF
RPH = 512
F = HEADS * HID


def _mm_body(x_ref, w_ref, b_ref, o_ref, *, act_slope):
    acc = jnp.dot(x_ref[...], w_ref[...], preferred_element_type=jnp.float32)
    acc = acc + b_ref[...]
    if act_slope is not None:
        acc = jnp.where(acc >= 0, acc, act_slope * acc)
    o_ref[...] = acc


def _mm(x, w, b, act_slope=None, block_rows=1024):
    m, k = x.shape
    nc = w.shape[1]
    grid = (m // block_rows,)
    return pl.pallas_call(
        functools.partial(_mm_body, act_slope=act_slope),
        grid=grid,
        in_specs=[
            pl.BlockSpec((block_rows, k), lambda i: (0, 0) if m == block_rows else (i, 0)),
            pl.BlockSpec((k, nc), lambda i: (0, 0)),
            pl.BlockSpec((1, nc), lambda i: (0, 0)),
        ],
        out_specs=pl.BlockSpec((block_rows, nc), lambda i: (0, 0) if m == block_rows else (i, 0)),
        out_shape=jax.ShapeDtypeStruct((m, nc), jnp.float32),
    )(x, w, b.reshape(1, nc))


def _h_and_scores_body(x_ref, w_ref, a2_ref, h_ref, sa_ref):
    h = jnp.dot(x_ref[...], w_ref[...], preferred_element_type=jnp.float32)
    h_ref[...] = h
    sa_ref[...] = jnp.dot(h, a2_ref[...], preferred_element_type=jnp.float32)


def _h_and_scores(x, w, a2, block_rows=2048):
    m, k = x.shape
    grid = (m // block_rows,)
    return pl.pallas_call(
        _h_and_scores_body,
        grid=grid,
        in_specs=[
            pl.BlockSpec((block_rows, k), lambda i: (i, 0)),
            pl.BlockSpec((k, F), lambda i: (0, 0)),
            pl.BlockSpec((F, 8), lambda i: (0, 0)),
        ],
        out_specs=[
            pl.BlockSpec((block_rows, F), lambda i: (i, 0)),
            pl.BlockSpec((block_rows, 8), lambda i: (i, 0)),
        ],
        out_shape=[
            jax.ShapeDtypeStruct((m, F), jnp.float32),
            jax.ShapeDtypeStruct((m, 8), jnp.float32),
        ],
    )(x, w, a2)


def _rot_head_body(p_ref, w1_ref, b1_ref, w2_ref, b2_ref, o_ref):
    r = jnp.dot(p_ref[...], w1_ref[...], preferred_element_type=jnp.float32)
    r = r + b1_ref[...]
    r = jnp.where(r >= 0, r, 0.01 * r)
    o_ref[...] = jnp.dot(r, w2_ref[...], preferred_element_type=jnp.float32) + b2_ref[...]


def _rot_head(pooled, fc1_w, fc1_b, fc2_w, fc2_b):
    ncp = 8704
    w2 = jnp.zeros((F, ncp), jnp.float32).at[:, :ROT_OUT].set(fc2_w)
    b2 = jnp.zeros((ncp,), jnp.float32).at[:ROT_OUT].set(fc2_b)
    br = 512
    out = pl.pallas_call(
        _rot_head_body,
        grid=(B // br,),
        in_specs=[
            pl.BlockSpec((br, F), lambda i: (i, 0)),
            pl.BlockSpec((F, F), lambda i: (0, 0)),
            pl.BlockSpec((1, F), lambda i: (0, 0)),
            pl.BlockSpec((F, ncp), lambda i: (0, 0)),
            pl.BlockSpec((1, ncp), lambda i: (0, 0)),
        ],
        out_specs=pl.BlockSpec((br, ncp), lambda i: (i, 0)),
        out_shape=jax.ShapeDtypeStruct((B, ncp), jnp.float32),
    )(pooled, fc1_w, fc1_b.reshape(1, F), w2, b2.reshape(1, ncp))
    return out[:, :ROT_OUT]


def _root_head_body(rc_ref, w1_ref, b1_ref, w2_ref, b2_ref, w3_ref, b3_ref, o_ref):
    g = jnp.dot(rc_ref[...], w1_ref[...], preferred_element_type=jnp.float32) + b1_ref[...]
    g = jnp.where(g >= 0, g, 0.01 * g)
    g = jnp.dot(g, w2_ref[...], preferred_element_type=jnp.float32) + b2_ref[...]
    g = jnp.where(g >= 0, g, 0.01 * g)
    o_ref[...] = jnp.dot(g, w3_ref[...], preferred_element_type=jnp.float32) + b3_ref[...]


def _root_head(rc, rh1_w, rh1_b, rh2_w, rh2_b, rh3_w, rh3_b):
    per_graph = CL * GF
    bs = rc.shape[0] // per_graph
    rc = rc.reshape(bs, per_graph)
    no = TL * GF
    nop = 128
    w3 = jnp.zeros((RPH, nop), jnp.float32).at[:, :no].set(rh3_w)
    b3 = jnp.zeros((nop,), jnp.float32).at[:no].set(rh3_b)
    br = 1024
    out = pl.pallas_call(
        _root_head_body,
        grid=(bs // br,),
        in_specs=[
            pl.BlockSpec((br, per_graph), lambda i: (i, 0)),
            pl.BlockSpec((per_graph, RPH), lambda i: (0, 0)),
            pl.BlockSpec((1, RPH), lambda i: (0, 0)),
            pl.BlockSpec((RPH, RPH), lambda i: (0, 0)),
            pl.BlockSpec((1, RPH), lambda i: (0, 0)),
            pl.BlockSpec((RPH, nop), lambda i: (0, 0)),
            pl.BlockSpec((1, nop), lambda i: (0, 0)),
        ],
        out_specs=pl.BlockSpec((br, nop), lambda i: (i, 0)),
        out_shape=jax.ShapeDtypeStruct((bs, nop), jnp.float32),
    )(rc, rh1_w, rh1_b.reshape(1, RPH), rh2_w, rh2_b.reshape(1, RPH), w3, b3.reshape(1, nop))
    return out[:, :no]


def _gat_edges(h, sa, src, dst, b):
    n = h.shape[0]
    asrc, adst = sa[:, :HEADS], sa[:, HEADS:]
    alpha = asrc[src] + adst[dst]
    alpha = jnp.where(alpha >= 0, alpha, 0.2 * alpha)
    ex = jnp.exp(alpha)
    den = jax.ops.segment_sum(ex, dst, num_segments=n)
    w = ex / (den[dst] + 1e-16)
    hh = h.reshape(n, HEADS, HID)
    out = jax.ops.segment_sum(hh[src] * w[:, :, None], dst, num_segments=n)
    return out.reshape(n, F) + b


def kernel(x, edge_index, batch, root_ctx_norm,
           W0, a_s0, a_d0, b0,
           W1, a_s1, a_d1, b1,
           W2, a_s2, a_d2, b2,
           fc1_w, fc1_b, fc2_w, fc2_b,
           rh1_w, rh1_b, rh2_w, rh2_b, rh3_w, rh3_b):
    loops = jnp.arange(N, dtype=edge_index.dtype)
    src = jnp.concatenate([edge_index[0], loops])
    dst = jnp.concatenate([edge_index[1], loops])

    def pack_a(a_s, a_d):
        z = jnp.zeros((HEADS, HID, 2 * HEADS), jnp.float32)
        z = z.at[jnp.arange(HEADS), :, jnp.arange(HEADS)].set(a_s)
        z = z.at[jnp.arange(HEADS), :, HEADS + jnp.arange(HEADS)].set(a_d)
        return z.reshape(F, 2 * HEADS)

    h, sa = _h_and_scores(x, W0, pack_a(a_s0, a_d0))
    h = _gat_edges(h, sa, src, dst, b0)
    h = jnp.where(h >= 0, h, 0.01 * h)
    h, sa = _h_and_scores(h, W1, pack_a(a_s1, a_d1))
    h = _gat_edges(h, sa, src, dst, b1)
    h = jnp.where(h >= 0, h, 0.01 * h)
    h, sa = _h_and_scores(h, W2, pack_a(a_s2, a_d2))
    h = _gat_edges(h, sa, src, dst, b2)

    sums = jax.ops.segment_sum(h, batch, num_segments=B)
    cnt = jax.ops.segment_sum(jnp.ones((N,), jnp.float32), batch, num_segments=B)
    pooled = sums / jnp.maximum(cnt, 1.0)[:, None]

    rot = _rot_head(pooled, fc1_w, fc1_b, fc2_w, fc2_b).reshape(B, NJ, TL * NF)
    root = _root_head(root_ctx_norm.reshape(-1), rh1_w, rh1_b, rh2_w, rh2_b, rh3_w, rh3_b)
    return rot, root


# speedup vs baseline: 21.5727x; 20.6574x over previous
import dataclasses
import functools

import jax
import jax.numpy as jnp
from jax import lax
from jax.experimental import pallas as pl
from jax.experimental.pallas import tpu as pltpu
from jax.experimental.pallas import tpu_sc as plsc

N = 98304
E = 188416
B = 4096
HEADS = 4
HID = 64
NF = 24
NJ = 24
TL = 15
CL = 11
GF = 3
ROT_OUT = NJ * TL * NF
RPH = 512
F = HEADS * HID

EFULL = E + N
CH = 64
NR = 4096
NRANGES = N // NR
EPAD = EFULL + CH
BR = 2048


def _h_and_scores_body(x_ref, w_ref, as_ref, ad_ref, h_ref, ss_ref, sd_ref, *, act):
    x = x_ref[...]
    if act:
        x = jnp.where(x >= 0, x, 0.01 * x)
    h = jnp.dot(x, w_ref[...], preferred_element_type=jnp.float32)
    h_ref[...] = h
    ss_ref[...] = jnp.dot(h, as_ref[...], preferred_element_type=jnp.float32)
    sd_ref[...] = jnp.dot(h, ad_ref[...], preferred_element_type=jnp.float32)


def _h_and_scores(x, w, a16s, a16d, act, block_rows=2048):
    m, k = x.shape
    return pl.pallas_call(
        functools.partial(_h_and_scores_body, act=act),
        grid=(m // block_rows,),
        in_specs=[
            pl.BlockSpec((block_rows, k), lambda i: (i, 0)),
            pl.BlockSpec((k, F), lambda i: (0, 0)),
            pl.BlockSpec((F, 16), lambda i: (0, 0)),
            pl.BlockSpec((F, 16), lambda i: (0, 0)),
        ],
        out_specs=[
            pl.BlockSpec((block_rows, F), lambda i: (i, 0)),
            pl.BlockSpec((block_rows, 16), lambda i: (i, 0)),
            pl.BlockSpec((block_rows, 16), lambda i: (i, 0)),
        ],
        out_shape=[
            jax.ShapeDtypeStruct((m, F), jnp.float32),
            jax.ShapeDtypeStruct((m, 16), jnp.float32),
            jax.ShapeDtypeStruct((m, 16), jnp.float32),
        ],
    )(x, w, a16s, a16d)


def _rot_head_body(p_ref, w1_ref, b1_ref, w2_ref, b2_ref, o_ref):
    r = jnp.dot(p_ref[...], w1_ref[...], preferred_element_type=jnp.float32)
    r = r + b1_ref[...]
    r = jnp.where(r >= 0, r, 0.01 * r)
    o_ref[...] = jnp.dot(r, w2_ref[...], preferred_element_type=jnp.float32) + b2_ref[...]


def _rot_head(pooled, fc1_w, fc1_b, fc2_w, fc2_b):
    ncp = 8704
    w2 = jnp.zeros((F, ncp), jnp.float32).at[:, :ROT_OUT].set(fc2_w)
    b2 = jnp.zeros((ncp,), jnp.float32).at[:ROT_OUT].set(fc2_b)
    br = 512
    out = pl.pallas_call(
        _rot_head_body,
        grid=(B // br,),
        in_specs=[
            pl.BlockSpec((br, F), lambda i: (i, 0)),
            pl.BlockSpec((F, F), lambda i: (0, 0)),
            pl.BlockSpec((1, F), lambda i: (0, 0)),
            pl.BlockSpec((F, ncp), lambda i: (0, 0)),
            pl.BlockSpec((1, ncp), lambda i: (0, 0)),
        ],
        out_specs=pl.BlockSpec((br, ncp), lambda i: (i, 0)),
        out_shape=jax.ShapeDtypeStruct((B, ncp), jnp.float32),
    )(pooled, fc1_w, fc1_b.reshape(1, F), w2, b2.reshape(1, ncp))
    return out[:, :ROT_OUT]


def _root_head_body(rc_ref, w1_ref, b1_ref, w2_ref, b2_ref, w3_ref, b3_ref, o_ref):
    g = jnp.dot(rc_ref[...], w1_ref[...], preferred_element_type=jnp.float32) + b1_ref[...]
    g = jnp.where(g >= 0, g, 0.01 * g)
    g = jnp.dot(g, w2_ref[...], preferred_element_type=jnp.float32) + b2_ref[...]
    g = jnp.where(g >= 0, g, 0.01 * g)
    o_ref[...] = jnp.dot(g, w3_ref[...], preferred_element_type=jnp.float32) + b3_ref[...]


def _root_head(rc, rh1_w, rh1_b, rh2_w, rh2_b, rh3_w, rh3_b):
    per_graph = CL * GF
    bs = rc.shape[0] // per_graph
    rc = rc.reshape(bs, per_graph)
    no = TL * GF
    nop = 128
    w3 = jnp.zeros((RPH, nop), jnp.float32).at[:, :no].set(rh3_w)
    b3 = jnp.zeros((nop,), jnp.float32).at[:no].set(rh3_b)
    br = 1024
    out = pl.pallas_call(
        _root_head_body,
        grid=(bs // br,),
        in_specs=[
            pl.BlockSpec((br, per_graph), lambda i: (i, 0)),
            pl.BlockSpec((per_graph, RPH), lambda i: (0, 0)),
            pl.BlockSpec((1, RPH), lambda i: (0, 0)),
            pl.BlockSpec((RPH, RPH), lambda i: (0, 0)),
            pl.BlockSpec((1, RPH), lambda i: (0, 0)),
            pl.BlockSpec((RPH, nop), lambda i: (0, 0)),
            pl.BlockSpec((1, nop), lambda i: (0, 0)),
        ],
        out_specs=pl.BlockSpec((br, nop), lambda i: (i, 0)),
        out_shape=jax.ShapeDtypeStruct((bs, nop), jnp.float32),
    )(rc, rh1_w, rh1_b.reshape(1, RPH), rh2_w, rh2_b.reshape(1, RPH), w3, b3.reshape(1, nop))
    return out[:, :no]


_MESH = plsc.VectorSubcoreMesh(core_axis_name="c", subcore_axis_name="s")


def _sc_params():
    cp = pltpu.CompilerParams()
    cp = dataclasses.replace(cp, needs_layout_passes=False,
                             use_tc_tiling_on_sc=False)
    return cp


def _lane():
    return lax.broadcasted_iota(jnp.int32, (16,), 0)


def _bcast_lane(vec, lane_idx):
    idx = jnp.full((16, 1), lane_idx, jnp.int32)
    dnums = lax.GatherDimensionNumbers(
        offset_dims=(), collapsed_slice_dims=(0,), start_index_map=(0,))
    return lax.gather(vec, idx, dnums, slice_sizes=(1,),
                      mode=lax.GatherScatterMode.PROMISE_IN_BOUNDS)


def _rp_at(rp_ref, i):
    lane = _lane()
    r0 = rp_ref[pl.ds(0, 16)]
    r1 = rp_ref[pl.ds(16, 16)]
    reg = jnp.where(jnp.full((16,), i, jnp.int32) < 16, r0, r1)
    v = jnp.where(lane == (i % 16), reg, 0)
    return jnp.sum(v)


def _gat_edge_kernel(h_hbm, ss_hbm, sd_hbm, src_hbm, dst_hbm, rp_hbm, b_hbm,
                     out_hbm,
                     srcb, dstb, dlocb, sasb, sadb, exb, rows, fbuf, dbuf,
                     zbuf, zbuf16, bbuf, rp_v,
                     acc_sh, den_sh):
    core = lax.axis_index("c")
    sub = lax.axis_index("s")

    pltpu.sync_copy(rp_hbm, rp_v)
    pltpu.sync_copy(b_hbm, bbuf)

    @pl.loop(0, CH)
    def _(i):
        for q in range(F // 16):
            zbuf[i, pl.ds(q * 16, 16)] = jnp.zeros((16,), jnp.float32)
        zbuf16[i, :] = jnp.zeros((16,), jnp.float32)

    @pl.loop(0, NRANGES // 2)
    def _(ri):
        r = ri * 2 + core
        r0 = r * NR
        row0 = sub * (NR // 16)
        for blk in range(NR // 16 // CH):
            pltpu.sync_copy(zbuf, acc_sh.at[pl.ds(row0 + blk * CH, CH)])
            pltpu.sync_copy(zbuf16, den_sh.at[pl.ds(row0 + blk * CH, CH)])
        plsc.subcore_barrier()

        p0 = _rp_at(rp_v, r)
        p1 = _rp_at(rp_v, r + 1)
        base0 = (p0 // 8) * 8
        nch = (p1 - base0 + CH - 1) // CH
        nloc = lax.max(0, (nch - sub + 15) // 16)

        @pl.loop(0, nloc)
        def _(k):
            base = base0 + (sub + k * 16) * CH
            pltpu.sync_copy(src_hbm.at[pl.ds(base, CH)], srcb)
            pltpu.sync_copy(dst_hbm.at[pl.ds(base, CH)], dstb)
            for t in range(CH // 16):
                d16 = dstb[pl.ds(t * 16, 16)]
                pos = base + t * 16 + _lane()
                valid = (pos >= p0) & (pos < p1)
                dlocb[pl.ds(t * 16, 16)] = jnp.where(valid, d16 - r0, NR)
            pltpu.sync_copy(ss_hbm.at[srcb], sasb)
            pltpu.sync_copy(sd_hbm.at[dstb], sadb)
            pltpu.sync_copy(h_hbm.at[srcb], rows)

            @pl.loop(0, CH)
            def _(e):
                a = sasb[e, :] + sadb[e, :]
                a = jnp.where(a >= 0, a, 0.2 * a)
                ex = jnp.exp(a)
                exb[e, :] = ex
                for hd in range(HEADS):
                    bh = _bcast_lane(ex, hd)
                    for q in range(HID // 16):
                        col = hd * HID + q * 16
                        rows[e, pl.ds(col, 16)] = rows[e, pl.ds(col, 16)] * bh

            pltpu.sync_copy(rows, acc_sh.at[dlocb], add=True)
            pltpu.sync_copy(exb, den_sh.at[dlocb], add=True)

        plsc.subcore_barrier()

        for blk in range(NR // 16 // CH):
            rr = row0 + blk * CH
            pltpu.sync_copy(acc_sh.at[pl.ds(rr, CH)], fbuf)
            pltpu.sync_copy(den_sh.at[pl.ds(rr, CH)], dbuf)

            @pl.loop(0, CH)
            def _(i):
                rec = 1.0 / (dbuf[i, :] + 1e-16)
                for hd in range(HEADS):
                    bh = _bcast_lane(rec, hd)
                    for q in range(HID // 16):
                        col = hd * HID + q * 16
                        fbuf[i, pl.ds(col, 16)] = (
                            fbuf[i, pl.ds(col, 16)] * bh + bbuf[pl.ds(col, 16)])

            pltpu.sync_copy(fbuf, out_hbm.at[pl.ds(r0 + rr, CH)])
        plsc.subcore_barrier()


def _gat_edges_sc(h, ss, sd, srcp, dstp, rowptr, b):
    kfn = pl.kernel(
        _gat_edge_kernel,
        out_type=jax.ShapeDtypeStruct((N, F), jnp.float32),
        mesh=_MESH,
        scratch_types=[
            pltpu.VMEM((CH,), jnp.int32),
            pltpu.VMEM((CH,), jnp.int32),
            pltpu.VMEM((CH,), jnp.int32),
            pltpu.VMEM((CH, 16), jnp.float32),
            pltpu.VMEM((CH, 16), jnp.float32),
            pltpu.VMEM((CH, 16), jnp.float32),
            pltpu.VMEM((CH, F), jnp.float32),
            pltpu.VMEM((CH, F), jnp.float32),
            pltpu.VMEM((CH, 16), jnp.float32),
            pltpu.VMEM((CH, F), jnp.float32),
            pltpu.VMEM((CH, 16), jnp.float32),
            pltpu.VMEM((F,), jnp.float32),
            pltpu.VMEM((32,), jnp.int32),
            pltpu.VMEM_SHARED((NR + 8, F), jnp.float32),
            pltpu.VMEM_SHARED((NR + 8, 16), jnp.float32),
        ],
        compiler_params=_sc_params(),
    )
    return kfn(h, ss, sd, srcp, dstp, rowptr, b)


def _pool_kernel(h_hbm, bat_hbm, rp_hbm, out_hbm,
                 batb, blocb, rows, onesb, fbuf, cbuf, zbuf, zbuf16, rp_v,
                 acc_sh, cnt_sh):
    core = lax.axis_index("c")
    sub = lax.axis_index("s")
    pltpu.sync_copy(rp_hbm, rp_v)

    @pl.loop(0, CH)
    def _(i):
        for q in range(F // 16):
            zbuf[i, pl.ds(q * 16, 16)] = jnp.zeros((16,), jnp.float32)
        zbuf16[i, :] = jnp.zeros((16,), jnp.float32)
        onesb[i, :] = jnp.ones((16,), jnp.float32)

    g0 = core * BR
    row0 = sub * (BR // 16)
    for blk in range(BR // 16 // CH):
        pltpu.sync_copy(zbuf, acc_sh.at[pl.ds(row0 + blk * CH, CH)])
        pltpu.sync_copy(zbuf16, cnt_sh.at[pl.ds(row0 + blk * CH, CH)])
    plsc.subcore_barrier()

    p0 = _rp_at(rp_v, core)
    p1 = _rp_at(rp_v, core + 1)
    base0 = (p0 // 8) * 8
    nch = (p1 - base0 + CH - 1) // CH
    nloc = lax.max(0, (nch - sub + 15) // 16)

    @pl.loop(0, nloc)
    def _(k):
        base = base0 + (sub + k * 16) * CH
        pltpu.sync_copy(bat_hbm.at[pl.ds(base, CH)], batb)
        for t in range(CH // 16):
            b16 = batb[pl.ds(t * 16, 16)]
            pos = base + t * 16 + _lane()
            valid = (pos >= p0) & (pos < p1)
            blocb[pl.ds(t * 16, 16)] = jnp.where(valid, b16 - g0, BR)
        pltpu.sync_copy(h_hbm.at[pl.ds(base, CH)], rows)
        pltpu.sync_copy(rows, acc_sh.at[blocb], add=True)
        pltpu.sync_copy(onesb, cnt_sh.at[blocb], add=True)

    plsc.subcore_barrier()

    for blk in range(BR // 16 // CH):
        rr = row0 + blk * CH
        pltpu.sync_copy(acc_sh.at[pl.ds(rr, CH)], fbuf)
        pltpu.sync_copy(cnt_sh.at[pl.ds(rr, CH)], cbuf)

        @pl.loop(0, CH)
        def _(i):
            rec = 1.0 / jnp.maximum(cbuf[i, :], 1.0)
            bh = _bcast_lane(rec, 0)
            for q in range(F // 16):
                col = q * 16
                fbuf[i, pl.ds(col, 16)] = fbuf[i, pl.ds(col, 16)] * bh

        pltpu.sync_copy(fbuf, out_hbm.at[pl.ds(g0 + rr, CH)])


def _pool_sc(h, batch_p, rowptr_b):
    kfn = pl.kernel(
        _pool_kernel,
        out_type=jax.ShapeDtypeStruct((B, F), jnp.float32),
        mesh=_MESH,
        scratch_types=[
            pltpu.VMEM((CH,), jnp.int32),
            pltpu.VMEM((CH,), jnp.int32),
            pltpu.VMEM((CH, F), jnp.float32),
            pltpu.VMEM((CH, 16), jnp.float32),
            pltpu.VMEM((CH, F), jnp.float32),
            pltpu.VMEM((CH, 16), jnp.float32),
            pltpu.VMEM((CH, F), jnp.float32),
            pltpu.VMEM((CH, 16), jnp.float32),
            pltpu.VMEM((32,), jnp.int32),
            pltpu.VMEM_SHARED((BR + 8, F), jnp.float32),
            pltpu.VMEM_SHARED((BR + 8, 16), jnp.float32),
        ],
        compiler_params=_sc_params(),
    )
    return kfn(h, batch_p, rowptr_b)


def kernel(x, edge_index, batch, root_ctx_norm,
           W0, a_s0, a_d0, b0,
           W1, a_s1, a_d1, b1,
           W2, a_s2, a_d2, b2,
           fc1_w, fc1_b, fc2_w, fc2_b,
           rh1_w, rh1_b, rh2_w, rh2_b, rh3_w, rh3_b):
    loops = jnp.arange(N, dtype=jnp.int32)
    src = jnp.concatenate([edge_index[0].astype(jnp.int32), loops])
    dst = jnp.concatenate([edge_index[1].astype(jnp.int32), loops])
    dst_s, src_s = lax.sort((dst, src), num_keys=1)
    srcp = jnp.zeros((EPAD,), jnp.int32).at[:EFULL].set(src_s)
    dstp = jnp.zeros((EPAD,), jnp.int32).at[:EFULL].set(dst_s)
    rowptr = jnp.zeros((32,), jnp.int32).at[:NRANGES + 1].set(
        jnp.searchsorted(
            dst_s, jnp.arange(0, N + 1, NR, dtype=jnp.int32)).astype(jnp.int32))
    batch32 = batch.astype(jnp.int32)
    rowptr_b = jnp.zeros((32,), jnp.int32).at[:3].set(
        jnp.searchsorted(
            batch32, jnp.arange(0, B + 1, BR, dtype=jnp.int32)).astype(jnp.int32))

    def pack_a16(a):
        z = jnp.zeros((HEADS, HID, 16), jnp.float32)
        z = z.at[jnp.arange(HEADS), :, jnp.arange(HEADS)].set(a)
        return z.reshape(F, 16)

    h, ss, sd = _h_and_scores(x, W0, pack_a16(a_s0), pack_a16(a_d0), act=False)
    h = _gat_edges_sc(h, ss, sd, srcp, dstp, rowptr, b0)
    h, ss, sd = _h_and_scores(h, W1, pack_a16(a_s1), pack_a16(a_d1), act=True)
    h = _gat_edges_sc(h, ss, sd, srcp, dstp, rowptr, b1)
    h, ss, sd = _h_and_scores(h, W2, pack_a16(a_s2), pack_a16(a_d2), act=True)
    h = _gat_edges_sc(h, ss, sd, srcp, dstp, rowptr, b2)

    pooled = _pool_sc(h, batch32, rowptr_b)

    rot = _rot_head(pooled, fc1_w, fc1_b, fc2_w, fc2_b).reshape(B, NJ, TL * NF)
    root = _root_head(root_ctx_norm.reshape(-1), rh1_w, rh1_b, rh2_w, rh2_b, rh3_w, rh3_b)
    return rot, root
